# baseline (speedup 1.0000x reference)
import functools

import jax
import jax.numpy as jnp
from jax import lax
from jax.experimental import pallas as pl
from jax.experimental.pallas import tpu as pltpu
from jax.experimental.pallas import tpu_sc as plsc

NUM_ROWS = 1000000
DIM = 64
BATCH = 4096
SEQ = 200
LANES = 128
NBUF = 4
SPLIT = 499968
TBL_H = 500096

_INFO = plsc.get_sparse_core_info()
NC = _INFO.num_cores
NS = _INFO.num_subcores
NW = NC * NS
NGRP = SEQ // NBUF


@functools.partial(
    pl.kernel,
    mesh=plsc.VectorSubcoreMesh(core_axis_name="c", subcore_axis_name="s"),
    compiler_params=pltpu.CompilerParams(
        use_tc_tiling_on_sc=True,
        needs_layout_passes=False,
        skip_device_barrier=True,
    ),
    out_type=jax.ShapeDtypeStruct((SEQ, DIM, BATCH), jnp.float32),
    scratch_types=[
        pltpu.VMEM((SEQ, LANES), jnp.int32),
    ]
    + [pltpu.VMEM((LANES,), jnp.int32) for _ in range(NBUF)]
    + [pltpu.VMEM((LANES, LANES), jnp.float32) for _ in range(NBUF)]
    + [pltpu.VMEM((DIM, LANES), jnp.float32) for _ in range(NBUF)]
    + [
        pltpu.SemaphoreType.DMA,
        pltpu.SemaphoreType.DMA,
    ],
)
def _emb_lookup(idx_hbm, table_hbm, out_hbm, idx_v, *rest):
    pidx = rest[0:NBUF]
    rows = rest[NBUF:2 * NBUF]
    obuf = rest[2 * NBUF:3 * NBUF]
    gsem, ssem = rest[3 * NBUF], rest[3 * NBUF + 1]

    wid = lax.axis_index("s") * NC + lax.axis_index("c")
    obase = wid * LANES

    pltpu.sync_copy(idx_hbm.at[:, pl.ds(obase, LANES)], idx_v)

    lane16 = lax.iota(jnp.int32, 16)

    def compute_pidx(k, t):
        for lg in range(LANES // 16):
            v = idx_v[t, pl.ds(lg * 16, 16)]
            m = (v >= SPLIT).astype(jnp.int32)
            pidx[k][pl.ds(lg * 16, 16)] = v - m * SPLIT

    def transpose_extract(k, t):
        for lg in range(LANES // 16):
            v = idx_v[t, pl.ds(lg * 16, 16)]
            half16 = (v >= SPLIT).astype(jnp.int32) << 6
            row_idx = lane16 + (lg * 16)

            @plsc.parallel_loop(0, DIM, unroll=8)
            def _(c):
                d_vec = (lane16 + c) & (DIM - 1)
                vals = plsc.load_gather(rows[k], [row_idx, half16 + d_vec])
                plsc.store_scatter(obuf[k], [d_vec, row_idx], vals)

    def body(g, _):
        t0 = g * NBUF
        handles = []
        for k in range(NBUF):
            compute_pidx(k, t0 + k)
            handles.append(
                pltpu.async_copy(table_hbm.at[pidx[k]], rows[k], gsem)
            )
        for k in range(NBUF):
            @pl.when(g > 0)
            def _():
                pltpu.make_async_copy(
                    obuf[k],
                    out_hbm.at[t0 + k - NBUF, :, pl.ds(obase, LANES)],
                    ssem,
                ).wait()

            handles[k].wait()
            transpose_extract(k, t0 + k)
            pltpu.async_copy(
                obuf[k], out_hbm.at[t0 + k, :, pl.ds(obase, LANES)], ssem
            )
        return 0

    lax.fori_loop(0, NGRP, body, 0)
    for k in range(NBUF):
        pltpu.make_async_copy(
            obuf[k], out_hbm.at[SEQ - NBUF + k, :, pl.ds(obase, LANES)], ssem
        ).wait()


def _pair_table_body(lo_ref, hi_ref, out_ref):
    out_ref[...] = jnp.concatenate(
        [jnp.transpose(lo_ref[...]), jnp.transpose(hi_ref[...])], axis=1
    )


_N_TBLK = TBL_H // (2 * DIM)


def _pair_table(wt):
    return pl.pallas_call(
        _pair_table_body,
        grid=(_N_TBLK,),
        in_specs=[
            pl.BlockSpec((DIM, 2 * DIM), lambda g: (0, g)),
            pl.BlockSpec((DIM, 2 * DIM), lambda g: (0, g + SPLIT // (2 * DIM))),
        ],
        out_specs=pl.BlockSpec((2 * DIM, 2 * DIM), lambda g: (g, 0)),
        out_shape=jax.ShapeDtypeStruct((TBL_H, 2 * DIM), jnp.float32),
    )(wt, wt)


def kernel(token_ids, weight):
    idx = token_ids.astype(jnp.int32).T
    table2 = _pair_table(weight.T)
    out = _emb_lookup(idx, table2)
    return out.transpose(2, 0, 1)


# speedup vs baseline: 4.4425x; 4.4425x over previous
import functools

import jax
import jax.numpy as jnp
from jax import lax
from jax.experimental import pallas as pl
from jax.experimental.pallas import tpu as pltpu
from jax.experimental.pallas import tpu_sc as plsc

NUM_ROWS = 1000000
DIM = 64
BATCH = 4096
SEQ = 200
LANES = 128
NBUF = 4
SPLIT = 499968
TBLK_W = 2688
TBL_H = 187 * TBLK_W

_INFO = plsc.get_sparse_core_info()
NC = _INFO.num_cores
NS = _INFO.num_subcores
NW = NC * NS
NGRP = SEQ // NBUF


@functools.partial(
    pl.kernel,
    mesh=plsc.VectorSubcoreMesh(core_axis_name="c", subcore_axis_name="s"),
    compiler_params=pltpu.CompilerParams(
        use_tc_tiling_on_sc=True,
        needs_layout_passes=False,
        skip_device_barrier=True,
    ),
    out_type=jax.ShapeDtypeStruct((SEQ, DIM, BATCH), jnp.float32),
    scratch_types=[
        pltpu.VMEM((SEQ, LANES), jnp.int32),
    ]
    + [pltpu.VMEM((LANES,), jnp.int32) for _ in range(NBUF)]
    + [pltpu.VMEM((LANES, LANES), jnp.float32) for _ in range(NBUF)]
    + [pltpu.VMEM((DIM, LANES), jnp.float32) for _ in range(NBUF)]
    + [
        pltpu.SemaphoreType.DMA,
        pltpu.SemaphoreType.DMA,
    ],
)
def _emb_lookup(idx_hbm, table_hbm, out_hbm, idx_v, *rest):
    pidx = rest[0:NBUF]
    rows = rest[NBUF:2 * NBUF]
    obuf = rest[2 * NBUF:3 * NBUF]
    gsem, ssem = rest[3 * NBUF], rest[3 * NBUF + 1]

    wid = lax.axis_index("s") * NC + lax.axis_index("c")
    obase = wid * LANES

    pltpu.sync_copy(idx_hbm.at[:, pl.ds(obase, LANES)], idx_v)

    lane16 = lax.iota(jnp.int32, 16)

    def compute_pidx(k, t):
        for lg in range(LANES // 16):
            v = idx_v[t, pl.ds(lg * 16, 16)]
            m = (v >= SPLIT).astype(jnp.int32)
            pidx[k][pl.ds(lg * 16, 16)] = v - m * SPLIT

    def transpose_extract(k, t):
        for lg in range(LANES // 16):
            v = idx_v[t, pl.ds(lg * 16, 16)]
            half16 = (v >= SPLIT).astype(jnp.int32) << 6
            row_idx = lane16 + (lg * 16)

            @plsc.parallel_loop(0, DIM, unroll=8)
            def _(c):
                d_vec = (lane16 + c) & (DIM - 1)
                vals = plsc.load_gather(rows[k], [row_idx, half16 + d_vec])
                plsc.store_scatter(obuf[k], [d_vec, row_idx], vals)

    def body(g, _):
        t0 = g * NBUF
        handles = []
        for k in range(NBUF):
            compute_pidx(k, t0 + k)
            handles.append(
                pltpu.async_copy(table_hbm.at[pidx[k]], rows[k], gsem)
            )
        for k in range(NBUF):
            @pl.when(g > 0)
            def _():
                pltpu.make_async_copy(
                    obuf[k],
                    out_hbm.at[t0 + k - NBUF, :, pl.ds(obase, LANES)],
                    ssem,
                ).wait()

            handles[k].wait()
            transpose_extract(k, t0 + k)
            pltpu.async_copy(
                obuf[k], out_hbm.at[t0 + k, :, pl.ds(obase, LANES)], ssem
            )
        return 0

    lax.fori_loop(0, NGRP, body, 0)
    for k in range(NBUF):
        pltpu.make_async_copy(
            obuf[k], out_hbm.at[SEQ - NBUF + k, :, pl.ds(obase, LANES)], ssem
        ).wait()


def _pair_table_body(lo_ref, hi_ref, out_ref):
    out_ref[...] = jnp.concatenate(
        [jnp.transpose(lo_ref[...]), jnp.transpose(hi_ref[...])], axis=1
    )


_N_TBLK = TBL_H // TBLK_W


def _pair_table(wt):
    return pl.pallas_call(
        _pair_table_body,
        grid=(_N_TBLK,),
        in_specs=[
            pl.BlockSpec((DIM, TBLK_W), lambda g: (0, g)),
            pl.BlockSpec((DIM, TBLK_W), lambda g: (0, g + SPLIT // TBLK_W)),
        ],
        out_specs=pl.BlockSpec((TBLK_W, 2 * DIM), lambda g: (g, 0)),
        out_shape=jax.ShapeDtypeStruct((TBL_H, 2 * DIM), jnp.float32),
    )(wt, wt)


def kernel(token_ids, weight):
    idx = token_ids.astype(jnp.int32).T
    table2 = _pair_table(weight.T)
    out = _emb_lookup(idx, table2)
    return out.transpose(2, 0, 1)


# speedup vs baseline: 4.8877x; 1.1002x over previous
import functools

import jax
import jax.numpy as jnp
from jax import lax
from jax.experimental import pallas as pl
from jax.experimental.pallas import tpu as pltpu
from jax.experimental.pallas import tpu_sc as plsc

NUM_ROWS = 1000000
DIM = 64
BATCH = 4096
SEQ = 200
LANES = 128
NBUF = 4
SPLIT = 499968
TBLK_W = 5376
TBL_H = 94 * TBLK_W

_INFO = plsc.get_sparse_core_info()
NC = _INFO.num_cores
NS = _INFO.num_subcores
NW = NC * NS
NGRP = SEQ // NBUF


@functools.partial(
    pl.kernel,
    mesh=plsc.VectorSubcoreMesh(core_axis_name="c", subcore_axis_name="s"),
    compiler_params=pltpu.CompilerParams(
        use_tc_tiling_on_sc=True,
        needs_layout_passes=False,
        skip_device_barrier=True,
    ),
    out_type=jax.ShapeDtypeStruct((SEQ, DIM, BATCH), jnp.float32),
    scratch_types=[
        pltpu.VMEM((SEQ, LANES), jnp.int32),
    ]
    + [pltpu.VMEM((LANES,), jnp.int32) for _ in range(NBUF)]
    + [pltpu.VMEM((LANES, LANES), jnp.float32) for _ in range(NBUF)]
    + [pltpu.VMEM((DIM, LANES), jnp.float32) for _ in range(NBUF)]
    + [
        pltpu.SemaphoreType.DMA,
        pltpu.SemaphoreType.DMA,
    ],
)
def _emb_lookup(idx_hbm, table_hbm, out_hbm, idx_v, *rest):
    pidx = rest[0:NBUF]
    rows = rest[NBUF:2 * NBUF]
    obuf = rest[2 * NBUF:3 * NBUF]
    gsem, ssem = rest[3 * NBUF], rest[3 * NBUF + 1]

    wid = lax.axis_index("s") * NC + lax.axis_index("c")
    obase = wid * LANES

    pltpu.sync_copy(idx_hbm.at[:, pl.ds(obase, LANES)], idx_v)

    lane16 = lax.iota(jnp.int32, 16)

    def compute_pidx(k, t):
        for lg in range(LANES // 16):
            v = idx_v[t, pl.ds(lg * 16, 16)]
            m = (v >= SPLIT).astype(jnp.int32)
            pidx[k][pl.ds(lg * 16, 16)] = v - m * SPLIT

    def transpose_extract(k, t):
        for lg in range(LANES // 16):
            v = idx_v[t, pl.ds(lg * 16, 16)]
            half16 = (v >= SPLIT).astype(jnp.int32) << 6
            row_idx = lane16 + (lg * 16)

            @plsc.parallel_loop(0, DIM, unroll=8)
            def _(c):
                d_vec = (lane16 + c) & (DIM - 1)
                vals = plsc.load_gather(rows[k], [row_idx, half16 + d_vec])
                plsc.store_scatter(obuf[k], [d_vec, row_idx], vals)

    def body(g, _):
        t0 = g * NBUF
        handles = []
        for k in range(NBUF):
            compute_pidx(k, t0 + k)
            handles.append(
                pltpu.async_copy(table_hbm.at[pidx[k]], rows[k], gsem)
            )
        for k in range(NBUF):
            @pl.when(g > 0)
            def _():
                pltpu.make_async_copy(
                    obuf[k],
                    out_hbm.at[t0 + k - NBUF, :, pl.ds(obase, LANES)],
                    ssem,
                ).wait()

            handles[k].wait()
            transpose_extract(k, t0 + k)
            pltpu.async_copy(
                obuf[k], out_hbm.at[t0 + k, :, pl.ds(obase, LANES)], ssem
            )
        return 0

    lax.fori_loop(0, NGRP, body, 0)
    for k in range(NBUF):
        pltpu.make_async_copy(
            obuf[k], out_hbm.at[SEQ - NBUF + k, :, pl.ds(obase, LANES)], ssem
        ).wait()


def _pair_table_body(lo_ref, hi_ref, out_ref):
    out_ref[...] = jnp.concatenate(
        [jnp.transpose(lo_ref[...]), jnp.transpose(hi_ref[...])], axis=1
    )


_N_TBLK = TBL_H // TBLK_W


def _pair_table(wt):
    return pl.pallas_call(
        _pair_table_body,
        grid=(_N_TBLK,),
        in_specs=[
            pl.BlockSpec((DIM, TBLK_W), lambda g: (0, g)),
            pl.BlockSpec((DIM, TBLK_W), lambda g: (0, g + SPLIT // TBLK_W)),
        ],
        out_specs=pl.BlockSpec((TBLK_W, 2 * DIM), lambda g: (g, 0)),
        out_shape=jax.ShapeDtypeStruct((TBL_H, 2 * DIM), jnp.float32),
    )(wt, wt)


def kernel(token_ids, weight):
    idx = token_ids.astype(jnp.int32).T
    table2 = _pair_table(weight.T)
    out = _emb_lookup(idx, table2)
    return out.transpose(2, 0, 1)


# speedup vs baseline: 5.1489x; 1.0534x over previous
import functools

import jax
import jax.numpy as jnp
from jax import lax
from jax.experimental import pallas as pl
from jax.experimental.pallas import tpu as pltpu
from jax.experimental.pallas import tpu_sc as plsc

NUM_ROWS = 1000000
DIM = 64
BATCH = 4096
SEQ = 200
LANES = 128
NBUF = 4
SPLIT = 499968
TBLK_W = 11904
TBL_H = 43 * TBLK_W

_INFO = plsc.get_sparse_core_info()
NC = _INFO.num_cores
NS = _INFO.num_subcores
NW = NC * NS
NGRP = SEQ // NBUF


@functools.partial(
    pl.kernel,
    mesh=plsc.VectorSubcoreMesh(core_axis_name="c", subcore_axis_name="s"),
    compiler_params=pltpu.CompilerParams(
        use_tc_tiling_on_sc=True,
        needs_layout_passes=False,
        skip_device_barrier=True,
    ),
    out_type=jax.ShapeDtypeStruct((SEQ, DIM, BATCH), jnp.float32),
    scratch_types=[
        pltpu.VMEM((SEQ, LANES), jnp.int32),
    ]
    + [pltpu.VMEM((LANES,), jnp.int32) for _ in range(NBUF)]
    + [pltpu.VMEM((LANES, LANES), jnp.float32) for _ in range(NBUF)]
    + [pltpu.VMEM((DIM, LANES), jnp.float32) for _ in range(NBUF)]
    + [
        pltpu.SemaphoreType.DMA,
        pltpu.SemaphoreType.DMA,
    ],
)
def _emb_lookup(idx_hbm, table_hbm, out_hbm, idx_v, *rest):
    pidx = rest[0:NBUF]
    rows = rest[NBUF:2 * NBUF]
    obuf = rest[2 * NBUF:3 * NBUF]
    gsem, ssem = rest[3 * NBUF], rest[3 * NBUF + 1]

    wid = lax.axis_index("s") * NC + lax.axis_index("c")
    obase = wid * LANES

    pltpu.sync_copy(idx_hbm.at[:, pl.ds(obase, LANES)], idx_v)

    lane16 = lax.iota(jnp.int32, 16)

    def compute_pidx(k, t):
        for lg in range(LANES // 16):
            v = idx_v[t, pl.ds(lg * 16, 16)]
            m = (v >= SPLIT).astype(jnp.int32)
            pidx[k][pl.ds(lg * 16, 16)] = v - m * SPLIT

    def transpose_extract(k, t):
        for lg in range(LANES // 16):
            v = idx_v[t, pl.ds(lg * 16, 16)]
            half16 = (v >= SPLIT).astype(jnp.int32) << 6
            row_idx = lane16 + (lg * 16)

            @plsc.parallel_loop(0, DIM, unroll=8)
            def _(c):
                d_vec = (lane16 + c) & (DIM - 1)
                vals = plsc.load_gather(rows[k], [row_idx, half16 + d_vec])
                plsc.store_scatter(obuf[k], [d_vec, row_idx], vals)

    def body(g, _):
        t0 = g * NBUF
        handles = []
        for k in range(NBUF):
            compute_pidx(k, t0 + k)
            handles.append(
                pltpu.async_copy(table_hbm.at[pidx[k]], rows[k], gsem)
            )
        for k in range(NBUF):
            @pl.when(g > 0)
            def _():
                pltpu.make_async_copy(
                    obuf[k],
                    out_hbm.at[t0 + k - NBUF, :, pl.ds(obase, LANES)],
                    ssem,
                ).wait()

            handles[k].wait()
            transpose_extract(k, t0 + k)
            pltpu.async_copy(
                obuf[k], out_hbm.at[t0 + k, :, pl.ds(obase, LANES)], ssem
            )
        return 0

    lax.fori_loop(0, NGRP, body, 0)
    for k in range(NBUF):
        pltpu.make_async_copy(
            obuf[k], out_hbm.at[SEQ - NBUF + k, :, pl.ds(obase, LANES)], ssem
        ).wait()


def _pair_table_body(lo_ref, hi_ref, out_ref):
    out_ref[...] = jnp.concatenate(
        [jnp.transpose(lo_ref[...]), jnp.transpose(hi_ref[...])], axis=1
    )


_N_TBLK = TBL_H // TBLK_W


def _pair_table(wt):
    return pl.pallas_call(
        _pair_table_body,
        grid=(_N_TBLK,),
        in_specs=[
            pl.BlockSpec((DIM, TBLK_W), lambda g: (0, g)),
            pl.BlockSpec((DIM, TBLK_W), lambda g: (0, g + SPLIT // TBLK_W)),
        ],
        out_specs=pl.BlockSpec((TBLK_W, 2 * DIM), lambda g: (g, 0)),
        out_shape=jax.ShapeDtypeStruct((TBL_H, 2 * DIM), jnp.float32),
    )(wt, wt)


def kernel(token_ids, weight):
    idx = token_ids.astype(jnp.int32).T
    table2 = _pair_table(weight.T)
    out = _emb_lookup(idx, table2)
    return out.transpose(2, 0, 1)
